# baseline (speedup 1.0000x reference)
import functools
import math

import jax
import jax.numpy as jnp
from jax import lax
from jax.experimental import pallas as pl
from jax.experimental.pallas import tpu as pltpu
from jax.experimental.pallas import tpu_sc as plsc

N = 10000
E = 320000
HID = 128
EA = 16
NB = 64

_NC = 2
_NS = 16
_NW = _NC * _NS
_GRP = 128
_GPW = 80
_E_PAD = _NW * _GPW * _GRP
_NGRP = _E_PAD // _GRP
_IDX_ROWS = _NGRP + 8

_GRPG = 256
_GPWG = _E_PAD // (_NW * _GRPG)
_NGRPG = _E_PAD // _GRPG
_IDX_ROWS_G = _NGRPG + 8
_TW = HID + 16

_S3 = math.sqrt(3.0)
_INTERP = False


def _sigmoid(x):
    return 1.0 / (1.0 + jnp.exp(-x))


_NBLK = 2000


def _stage1a_body(nf_ref, posp_ref, db_ref, mf_ref, mcol_ref, wc1t_ref, bc1_ref,
                  wc2p_ref, bc2p_ref, a_ref, b_ref, mc_ref, psc_ref):
    nf = nf_ref[...]
    a_ref[...] = jnp.dot(
        nf, mf_ref[...], preferred_element_type=jnp.float32).astype(jnp.bfloat16)
    b_ref[...] = jnp.dot(
        nf, mcol_ref[...], preferred_element_type=jnp.float32).astype(jnp.bfloat16)

    hc = jnp.dot(nf, wc1t_ref[...], preferred_element_type=jnp.float32) + bc1_ref[...]
    hc = hc * _sigmoid(hc)
    mc_ref[...] = jnp.dot(hc, wc2p_ref[...], preferred_element_type=jnp.float32) + bc2p_ref[...]

    posp = posp_ref[...]
    db = db_ref[...]
    onehot = (db == lax.broadcasted_iota(jnp.int32, (1, NB), 1)).astype(jnp.float32)
    psc = lax.dot_general(onehot, posp, (((0,), (0,)), ((), ())),
                          preferred_element_type=jnp.float32)

    @pl.when(pl.program_id(0) == 0)
    def _init():
        psc_ref[...] = psc

    @pl.when(pl.program_id(0) != 0)
    def _acc():
        psc_ref[...] += psc


def _stage1b_body(posp_ref, db_ref, mc_ref, psc_ref, nsc_ref):
    psc = psc_ref[...]
    com = psc[:, 0:3] / jnp.maximum(psc[:, 3:4], 1.0)
    posp = posp_ref[...]
    db = db_ref[...]
    mc = mc_ref[...]
    onehot = (db == lax.broadcasted_iota(jnp.int32, (1, NB), 1)).astype(jnp.float32)
    percom = jnp.dot(onehot, com, preferred_element_type=jnp.float32)

    d = posp[:, 0:3] - percom
    dx = d[:, 0:1]
    dy = d[:, 1:2]
    dz = d[:, 2:3]
    r = jnp.sqrt(dx * dx + dy * dy + dz * dz)
    inv = 1.0 / jnp.maximum(r, 1e-12)
    ux = dx * inv
    uy = dy * inv
    uz = dz * inv
    m0 = mc[:, 0:1]
    m1 = mc[:, 1:2]
    m2 = mc[:, 2:3]
    zeros = jnp.zeros_like(m0)
    nsc_ref[...] = jnp.concatenate([
        m0,
        m1 * ux, m1 * uy, m1 * uz,
        m2 * (_S3 * ux * uz),
        m2 * (_S3 * ux * uy),
        m2 * (uy * uy - 0.5 * (ux * ux + uz * uz)),
        m2 * (_S3 * uy * uz),
        m2 * (0.5 * _S3 * (uz * uz - ux * ux)),
        zeros, zeros, zeros, zeros, zeros, zeros, zeros,
    ], axis=1)


def _stage1(nf, posp, db2, mf, mcol, wc1t, bc1v, wc2p, bc2p):
    grid = N // _NBLK
    A, B, MC, PSC = pl.pallas_call(
        _stage1a_body,
        grid=(grid,),
        in_specs=[
            pl.BlockSpec((_NBLK, HID), lambda i: (i, 0)),
            pl.BlockSpec((_NBLK, 4), lambda i: (i, 0)),
            pl.BlockSpec((_NBLK, 1), lambda i: (i, 0)),
            pl.BlockSpec((HID, HID), lambda i: (0, 0)),
            pl.BlockSpec((HID, HID), lambda i: (0, 0)),
            pl.BlockSpec((HID, HID), lambda i: (0, 0)),
            pl.BlockSpec((1, HID), lambda i: (0, 0)),
            pl.BlockSpec((HID, 8), lambda i: (0, 0)),
            pl.BlockSpec((1, 8), lambda i: (0, 0)),
        ],
        out_specs=[
            pl.BlockSpec((_NBLK, HID), lambda i: (i, 0)),
            pl.BlockSpec((_NBLK, HID), lambda i: (i, 0)),
            pl.BlockSpec((_NBLK, 8), lambda i: (i, 0)),
            pl.BlockSpec((NB, 4), lambda i: (0, 0)),
        ],
        out_shape=[
            jax.ShapeDtypeStruct((N, HID), jnp.bfloat16),
            jax.ShapeDtypeStruct((N, HID), jnp.bfloat16),
            jax.ShapeDtypeStruct((N, 8), jnp.float32),
            jax.ShapeDtypeStruct((NB, 4), jnp.float32),
        ],
        interpret=_INTERP,
    )(nf, posp, db2, mf, mcol, wc1t, bc1v, wc2p, bc2p)

    nsc = pl.pallas_call(
        _stage1b_body,
        grid=(grid,),
        in_specs=[
            pl.BlockSpec((_NBLK, 4), lambda i: (i, 0)),
            pl.BlockSpec((_NBLK, 1), lambda i: (i, 0)),
            pl.BlockSpec((_NBLK, 8), lambda i: (i, 0)),
            pl.BlockSpec((NB, 4), lambda i: (0, 0)),
        ],
        out_specs=pl.BlockSpec((_NBLK, 16), lambda i: (i, 0)),
        out_shape=jax.ShapeDtypeStruct((N, 16), jnp.float32),
        interpret=_INTERP,
    )(posp, db2, MC, PSC)
    return A, B, nsc


def _sc_gather_body(ta_hbm, tb_hbm, p16_hbm, np16_hbm, r2d_hbm, c2d_hbm,
                    g_hbm, s_hbm,
                    ridx, cidx, gb0, sb0, gb1, sb1, sg0, sg1, so0, so1):
    cid = lax.axis_index("c")
    sid = lax.axis_index("s")
    wid = cid * _NS + sid
    start_g = wid * _GPWG

    pltpu.sync_copy(r2d_hbm.at[pl.ds(start_g, _GPWG + 8)], ridx)
    pltpu.sync_copy(c2d_hbm.at[pl.ds(start_g, _GPWG + 8)], cidx)

    def base(c, gb, sb, sg):
        pltpu.async_copy(ta_hbm.at[ridx.at[c]], gb, sg)
        pltpu.async_copy(p16_hbm.at[ridx.at[c]], sb, sg)

    def addg(c, gb, sb, sg):
        pltpu.async_copy(tb_hbm.at[cidx.at[c]], gb, sg, add=True)
        pltpu.async_copy(np16_hbm.at[cidx.at[c]], sb, sg, add=True)

    def wait_g(gb, sb, sg):
        pltpu.make_async_copy(ta_hbm.at[pl.ds(0, _GRPG)], gb, sg).wait()
        pltpu.make_async_copy(p16_hbm.at[pl.ds(0, _GRPG)], sb, sg).wait()

    def wout(c, gb, sb, so):
        esl = pl.ds((start_g + c) * _GRPG, _GRPG)
        pltpu.async_copy(gb, g_hbm.at[esl], so)
        pltpu.async_copy(sb, s_hbm.at[esl], so)

    def wait_o(gb, sb, so):
        pltpu.make_async_copy(gb, g_hbm.at[pl.ds(0, _GRPG)], so).wait()
        pltpu.make_async_copy(sb, s_hbm.at[pl.ds(0, _GRPG)], so).wait()

    base(0, gb0, sb0, sg0)

    def loop_body(u, _):
        e = 2 * u
        wait_g(gb0, sb0, sg0)
        addg(e, gb0, sb0, sg0)

        @pl.when(u > 0)
        def _():
            wait_o(gb1, sb1, so1)

        base(e + 1, gb1, sb1, sg1)
        wait_g(gb0, sb0, sg0)
        wout(e, gb0, sb0, so0)
        wait_g(gb1, sb1, sg1)
        addg(e + 1, gb1, sb1, sg1)
        wait_o(gb0, sb0, so0)
        base(e + 2, gb0, sb0, sg0)
        wait_g(gb1, sb1, sg1)
        wout(e + 1, gb1, sb1, so1)
        return 0

    lax.fori_loop(0, _GPWG // 2, loop_body, 0)
    wait_g(gb0, sb0, sg0)
    wait_o(gb1, sb1, so1)


def _sc_gather(TA, TB, posp16, negp16, row2dg, col2dg):
    mesh = plsc.VectorSubcoreMesh(core_axis_name="c", subcore_axis_name="s",
                                  num_cores=_NC, num_subcores=_NS)
    f = functools.partial(
        pl.kernel,
        out_type=[
            jax.ShapeDtypeStruct((_E_PAD, HID), jnp.bfloat16),
            jax.ShapeDtypeStruct((_E_PAD, 16), jnp.float32),
        ],
        mesh=mesh,
        scratch_types=[
            pltpu.VMEM((_GPWG + 8, _GRPG), jnp.int32),
            pltpu.VMEM((_GPWG + 8, _GRPG), jnp.int32),
            pltpu.VMEM((_GRPG, HID), jnp.bfloat16),
            pltpu.VMEM((_GRPG, 16), jnp.float32),
            pltpu.VMEM((_GRPG, HID), jnp.bfloat16),
            pltpu.VMEM((_GRPG, 16), jnp.float32),
            pltpu.SemaphoreType.DMA,
            pltpu.SemaphoreType.DMA,
            pltpu.SemaphoreType.DMA,
            pltpu.SemaphoreType.DMA,
        ],
        compiler_params=pltpu.CompilerParams(use_tc_tiling_on_sc=False),
    )(_sc_gather_body)
    return f(TA, TB, posp16, negp16, row2dg, col2dg)


def _sc_scatter_body(p_hbm, r2d_hbm, z_hbm, acc_hbm, pbuf, idxbuf, shared, sdma):
    cid = lax.axis_index("c")
    sid = lax.axis_index("s")
    wid = cid * _NS + sid

    @pl.when(sid == 0)
    def _():
        pltpu.sync_copy(z_hbm, shared)

    plsc.subcore_barrier()
    base_g = wid * _GPW

    def batch_body(b, _):
        gb = base_g + b * 16
        pltpu.sync_copy(r2d_hbm.at[pl.ds(gb, 16)], idxbuf)
        pltpu.sync_copy(p_hbm.at[pl.ds(gb * _GRP, 16 * _GRP)], pbuf)
        copies = []
        for j in range(16):
            copies.append(pltpu.async_copy(
                pbuf.at[pl.ds(j * _GRP, _GRP)], shared.at[idxbuf.at[j]],
                sdma, add=True))
        for c in copies:
            c.wait()
        return 0

    lax.fori_loop(0, _GPW // 16, batch_body, 0)
    plsc.subcore_barrier()

    @pl.when(sid == 0)
    def _():
        pltpu.sync_copy(shared, acc_hbm.at[cid])


def _sc_scatter(P, row2d, zeros_n16):
    mesh = plsc.VectorSubcoreMesh(core_axis_name="c", subcore_axis_name="s",
                                  num_cores=_NC, num_subcores=_NS)
    f = functools.partial(
        pl.kernel,
        out_type=jax.ShapeDtypeStruct((_NC, N, 16), jnp.float32),
        mesh=mesh,
        scratch_types=[
            pltpu.VMEM((16 * _GRP, 16), jnp.float32),
            pltpu.VMEM((16, _GRP), jnp.int32),
            pltpu.VMEM_SHARED((N, 16), jnp.float32),
            pltpu.SemaphoreType.DMA,
        ],
        compiler_params=pltpu.CompilerParams(use_tc_tiling_on_sc=False),
    )(_sc_scatter_body)
    return f(P, row2d, zeros_n16)


_EB = 4096


def _stage3_body(g_ref, ea_ref, s_ref, me_ref, b1_ref, wd_ref, w2p_ref, b2p_ref,
                 p_ref):
    g = g_ref[...].astype(jnp.float32)
    ea = ea_ref[...]
    st = s_ref[...].T
    dx = st[0:1, :]
    dy = st[1:2, :]
    dz = st[2:3, :]
    r2 = dx * dx + dy * dy + dz * dz
    rn = jnp.sqrt(r2)
    r_col = rn.T
    z = (g + jnp.dot(ea, me_ref[...], preferred_element_type=jnp.float32)
         + b1_ref[...] + r_col * wd_ref[...])
    h = z * _sigmoid(z)
    m = jnp.dot(h, w2p_ref[...], preferred_element_type=jnp.float32) + b2p_ref[...]
    mt = m.T
    inv = 1.0 / jnp.maximum(rn, 1e-12)
    ux = dx * inv
    uy = dy * inv
    uz = dz * inv
    m0 = mt[0:1, :]
    m1 = mt[1:2, :]
    m2 = mt[2:3, :]
    eid = pl.program_id(0) * _EB + lax.broadcasted_iota(jnp.int32, (1, _EB), 1)
    valid = (eid < E).astype(jnp.float32)
    zeros = jnp.zeros_like(m0)
    pt = jnp.concatenate([
        valid * m0,
        valid * (m1 * ux), valid * (m1 * uy), valid * (m1 * uz),
        valid * (m2 * (_S3 * ux * uz)),
        valid * (m2 * (_S3 * ux * uy)),
        valid * (m2 * (uy * uy - 0.5 * (ux * ux + uz * uz))),
        valid * (m2 * (_S3 * uy * uz)),
        valid * (m2 * (0.5 * _S3 * (uz * uz - ux * ux))),
        valid,
        zeros, zeros, zeros, zeros, zeros, zeros,
    ], axis=0)
    p_ref[...] = pt.T


def _stage3(G, ea, S, me, b1v, wdv, w2p, b2p):
    grid = _E_PAD // _EB
    return pl.pallas_call(
        _stage3_body,
        grid=(grid,),
        in_specs=[
            pl.BlockSpec((_EB, HID), lambda i: (i, 0)),
            pl.BlockSpec((_EB, EA), lambda i: (i, 0)),
            pl.BlockSpec((_EB, 16), lambda i: (i, 0)),
            pl.BlockSpec((EA, HID), lambda i: (0, 0)),
            pl.BlockSpec((1, HID), lambda i: (0, 0)),
            pl.BlockSpec((1, HID), lambda i: (0, 0)),
            pl.BlockSpec((HID, 8), lambda i: (0, 0)),
            pl.BlockSpec((1, 8), lambda i: (0, 0)),
        ],
        out_specs=pl.BlockSpec((_EB, 16), lambda i: (i, 0)),
        out_shape=jax.ShapeDtypeStruct((_E_PAD, 16), jnp.float32),
        interpret=_INTERP,
    )(G, ea, S, me, b1v, wdv, w2p, b2p)


def _stage5_body(acc0_ref, acc1_ref, nsc_ref, out_ref):
    t = acc0_ref[...] + acc1_ref[...]
    ns = t[:, 0:9] / jnp.maximum(t[:, 9:10], 1.0)
    out_ref[...] = ns + nsc_ref[:, 0:9]


def _stage5(acc0, acc1, nsc):
    return pl.pallas_call(
        _stage5_body,
        out_shape=jax.ShapeDtypeStruct((N, 9), jnp.float32),
        interpret=_INTERP,
    )(acc0, acc1, nsc)


def kernel(node_feat, node_pos, edge_index, edge_attr, data_batch,
           W1, b1, W2, b2, Wc1, bc1, Wc2, bc2):
    row = edge_index[0]
    col = edge_index[1]

    mf = W1[:, 1:1 + HID].T
    mcol = W1[:, 1 + HID:1 + 2 * HID].T
    me = W1[:, 1 + 2 * HID:].T
    wdv = W1[:, 0].reshape(1, HID)
    b1v = b1.reshape(1, HID)
    w2p = jnp.zeros((HID, 8), jnp.float32).at[:, 0:3].set(W2.T)
    b2p = jnp.zeros((1, 8), jnp.float32).at[0, 0:3].set(b2)
    wc1t = Wc1.T
    bc1v = bc1.reshape(1, HID)
    wc2p = jnp.zeros((HID, 8), jnp.float32).at[:, 0:3].set(Wc2.T)
    bc2p = jnp.zeros((1, 8), jnp.float32).at[0, 0:3].set(bc2)

    posp = jnp.concatenate(
        [node_pos, jnp.ones((N, 1), jnp.float32)], axis=1)
    db2 = data_batch.reshape(N, 1)

    TA, TB, nsc = _stage1(node_feat, posp, db2, mf, mcol, wc1t, bc1v, wc2p, bc2p)

    pad_rows = _IDX_ROWS - E // _GRP
    row2d = jnp.pad(row.reshape(E // _GRP, _GRP), ((0, pad_rows), (0, 0)))
    pad_rows_g = _IDX_ROWS_G - E // _GRPG
    row2dg = jnp.pad(row.reshape(E // _GRPG, _GRPG), ((0, pad_rows_g), (0, 0)))
    col2dg = jnp.pad(col.reshape(E // _GRPG, _GRPG), ((0, pad_rows_g), (0, 0)))
    eap = jnp.pad(edge_attr, ((0, _E_PAD - E), (0, 0)))
    posp16 = jnp.pad(node_pos, ((0, 0), (0, 13)))
    negp16 = jnp.pad(-node_pos, ((0, 0), (0, 13)))

    G, S = _sc_gather(TA, TB, posp16, negp16, row2dg, col2dg)

    P = _stage3(G, eap, S, me, b1v, wdv, w2p, b2p)

    acc = _sc_scatter(P, row2d, jnp.zeros((N, 16), jnp.float32))

    return _stage5(acc[0], acc[1], nsc)


# speedup vs baseline: 1.0500x; 1.0500x over previous
import functools
import math

import jax
import jax.numpy as jnp
from jax import lax
from jax.experimental import pallas as pl
from jax.experimental.pallas import tpu as pltpu
from jax.experimental.pallas import tpu_sc as plsc

N = 10000
E = 320000
HID = 128
EA = 16
NB = 64

_NC = 2
_NS = 16
_NW = _NC * _NS
_GRP = 128
_GPW = 80
_E_PAD = _NW * _GPW * _GRP
_NGRP = _E_PAD // _GRP
_IDX_ROWS = _NGRP + 8

_GRPG = 256
_GPWG = _E_PAD // (_NW * _GRPG)
_NGRPG = _E_PAD // _GRPG
_IDX_ROWS_G = _NGRPG + 8
_TW = HID + 16

_S3 = math.sqrt(3.0)
_INTERP = False


def _sigmoid(x):
    return 1.0 / (1.0 + jnp.exp(-x))


_NBLK = 2000


def _stage1a_body(nf_ref, posp_ref, db_ref, mf_ref, mcol_ref, wc1t_ref, bc1_ref,
                  wc2p_ref, bc2p_ref, a_ref, b_ref, mc_ref, psc_ref):
    nf = nf_ref[...]
    a_ref[...] = jnp.dot(nf, mf_ref[...], preferred_element_type=jnp.float32)
    b_ref[...] = jnp.dot(nf, mcol_ref[...], preferred_element_type=jnp.float32)

    hc = jnp.dot(nf, wc1t_ref[...], preferred_element_type=jnp.float32) + bc1_ref[...]
    hc = hc * _sigmoid(hc)
    mc_ref[...] = jnp.dot(hc, wc2p_ref[...], preferred_element_type=jnp.float32) + bc2p_ref[...]

    posp = posp_ref[...]
    db = db_ref[...]
    onehot = (db == lax.broadcasted_iota(jnp.int32, (1, NB), 1)).astype(jnp.float32)
    psc = lax.dot_general(onehot, posp, (((0,), (0,)), ((), ())),
                          preferred_element_type=jnp.float32)

    @pl.when(pl.program_id(0) == 0)
    def _init():
        psc_ref[...] = psc

    @pl.when(pl.program_id(0) != 0)
    def _acc():
        psc_ref[...] += psc


def _stage1b_body(posp_ref, db_ref, mc_ref, psc_ref, nsc_ref):
    psc = psc_ref[...]
    com = psc[:, 0:3] / jnp.maximum(psc[:, 3:4], 1.0)
    posp = posp_ref[...]
    db = db_ref[...]
    mc = mc_ref[...]
    onehot = (db == lax.broadcasted_iota(jnp.int32, (1, NB), 1)).astype(jnp.float32)
    percom = jnp.dot(onehot, com, preferred_element_type=jnp.float32)

    d = posp[:, 0:3] - percom
    dx = d[:, 0:1]
    dy = d[:, 1:2]
    dz = d[:, 2:3]
    r = jnp.sqrt(dx * dx + dy * dy + dz * dz)
    inv = 1.0 / jnp.maximum(r, 1e-12)
    ux = dx * inv
    uy = dy * inv
    uz = dz * inv
    m0 = mc[:, 0:1]
    m1 = mc[:, 1:2]
    m2 = mc[:, 2:3]
    zeros = jnp.zeros_like(m0)
    nsc_ref[...] = jnp.concatenate([
        m0,
        m1 * ux, m1 * uy, m1 * uz,
        m2 * (_S3 * ux * uz),
        m2 * (_S3 * ux * uy),
        m2 * (uy * uy - 0.5 * (ux * ux + uz * uz)),
        m2 * (_S3 * uy * uz),
        m2 * (0.5 * _S3 * (uz * uz - ux * ux)),
        zeros, zeros, zeros, zeros, zeros, zeros, zeros,
    ], axis=1)


def _stage1(nf, posp, db2, mf, mcol, wc1t, bc1v, wc2p, bc2p):
    grid = N // _NBLK
    A, B, MC, PSC = pl.pallas_call(
        _stage1a_body,
        grid=(grid,),
        in_specs=[
            pl.BlockSpec((_NBLK, HID), lambda i: (i, 0)),
            pl.BlockSpec((_NBLK, 4), lambda i: (i, 0)),
            pl.BlockSpec((_NBLK, 1), lambda i: (i, 0)),
            pl.BlockSpec((HID, HID), lambda i: (0, 0)),
            pl.BlockSpec((HID, HID), lambda i: (0, 0)),
            pl.BlockSpec((HID, HID), lambda i: (0, 0)),
            pl.BlockSpec((1, HID), lambda i: (0, 0)),
            pl.BlockSpec((HID, 8), lambda i: (0, 0)),
            pl.BlockSpec((1, 8), lambda i: (0, 0)),
        ],
        out_specs=[
            pl.BlockSpec((_NBLK, HID), lambda i: (i, 0)),
            pl.BlockSpec((_NBLK, HID), lambda i: (i, 0)),
            pl.BlockSpec((_NBLK, 8), lambda i: (i, 0)),
            pl.BlockSpec((NB, 4), lambda i: (0, 0)),
        ],
        out_shape=[
            jax.ShapeDtypeStruct((N, HID), jnp.float32),
            jax.ShapeDtypeStruct((N, HID), jnp.float32),
            jax.ShapeDtypeStruct((N, 8), jnp.float32),
            jax.ShapeDtypeStruct((NB, 4), jnp.float32),
        ],
        interpret=_INTERP,
    )(nf, posp, db2, mf, mcol, wc1t, bc1v, wc2p, bc2p)

    nsc = pl.pallas_call(
        _stage1b_body,
        grid=(grid,),
        in_specs=[
            pl.BlockSpec((_NBLK, 4), lambda i: (i, 0)),
            pl.BlockSpec((_NBLK, 1), lambda i: (i, 0)),
            pl.BlockSpec((_NBLK, 8), lambda i: (i, 0)),
            pl.BlockSpec((NB, 4), lambda i: (0, 0)),
        ],
        out_specs=pl.BlockSpec((_NBLK, 16), lambda i: (i, 0)),
        out_shape=jax.ShapeDtypeStruct((N, 16), jnp.float32),
        interpret=_INTERP,
    )(posp, db2, MC, PSC)
    return A, B, nsc


def _sc_gather_body(ta_hbm, tb_hbm, p16_hbm, np16_hbm, r2d_hbm, c2d_hbm,
                    g_hbm, s_hbm,
                    ridx, cidx, gb0, sb0, gb1, sb1, sg0, sg1, so0, so1):
    cid = lax.axis_index("c")
    sid = lax.axis_index("s")
    wid = cid * _NS + sid
    start_g = wid * _GPWG

    pltpu.sync_copy(r2d_hbm.at[pl.ds(start_g, _GPWG + 8)], ridx)
    pltpu.sync_copy(c2d_hbm.at[pl.ds(start_g, _GPWG + 8)], cidx)

    def base(c, gb, sb, sg):
        pltpu.async_copy(ta_hbm.at[ridx.at[c]], gb, sg)
        pltpu.async_copy(p16_hbm.at[ridx.at[c]], sb, sg)

    def addg(c, gb, sb, sg):
        pltpu.async_copy(tb_hbm.at[cidx.at[c]], gb, sg, add=True)
        pltpu.async_copy(np16_hbm.at[cidx.at[c]], sb, sg, add=True)

    def wait_g(gb, sb, sg):
        pltpu.make_async_copy(ta_hbm.at[pl.ds(0, _GRPG)], gb, sg).wait()
        pltpu.make_async_copy(p16_hbm.at[pl.ds(0, _GRPG)], sb, sg).wait()

    def wout(c, gb, sb, so):
        esl = pl.ds((start_g + c) * _GRPG, _GRPG)
        pltpu.async_copy(gb, g_hbm.at[esl], so)
        pltpu.async_copy(sb, s_hbm.at[esl], so)

    def wait_o(gb, sb, so):
        pltpu.make_async_copy(gb, g_hbm.at[pl.ds(0, _GRPG)], so).wait()
        pltpu.make_async_copy(sb, s_hbm.at[pl.ds(0, _GRPG)], so).wait()

    base(0, gb0, sb0, sg0)

    def loop_body(u, _):
        e = 2 * u
        wait_g(gb0, sb0, sg0)
        addg(e, gb0, sb0, sg0)

        @pl.when(u > 0)
        def _():
            wait_o(gb1, sb1, so1)

        base(e + 1, gb1, sb1, sg1)
        wait_g(gb0, sb0, sg0)
        wout(e, gb0, sb0, so0)
        wait_g(gb1, sb1, sg1)
        addg(e + 1, gb1, sb1, sg1)
        wait_o(gb0, sb0, so0)
        base(e + 2, gb0, sb0, sg0)
        wait_g(gb1, sb1, sg1)
        wout(e + 1, gb1, sb1, so1)
        return 0

    lax.fori_loop(0, _GPWG // 2, loop_body, 0)
    wait_g(gb0, sb0, sg0)
    wait_o(gb1, sb1, so1)


def _sc_gather(TA, TB, posp16, negp16, row2dg, col2dg):
    mesh = plsc.VectorSubcoreMesh(core_axis_name="c", subcore_axis_name="s",
                                  num_cores=_NC, num_subcores=_NS)
    f = functools.partial(
        pl.kernel,
        out_type=[
            jax.ShapeDtypeStruct((_E_PAD, HID), jnp.float32),
            jax.ShapeDtypeStruct((_E_PAD, 16), jnp.float32),
        ],
        mesh=mesh,
        scratch_types=[
            pltpu.VMEM((_GPWG + 8, _GRPG), jnp.int32),
            pltpu.VMEM((_GPWG + 8, _GRPG), jnp.int32),
            pltpu.VMEM((_GRPG, HID), jnp.float32),
            pltpu.VMEM((_GRPG, 16), jnp.float32),
            pltpu.VMEM((_GRPG, HID), jnp.float32),
            pltpu.VMEM((_GRPG, 16), jnp.float32),
            pltpu.SemaphoreType.DMA,
            pltpu.SemaphoreType.DMA,
            pltpu.SemaphoreType.DMA,
            pltpu.SemaphoreType.DMA,
        ],
        compiler_params=pltpu.CompilerParams(use_tc_tiling_on_sc=False),
    )(_sc_gather_body)
    return f(TA, TB, posp16, negp16, row2dg, col2dg)


def _sc_scatter_body(p_hbm, r2d_hbm, z_hbm, acc_hbm, pbuf, idxbuf, shared, sdma):
    cid = lax.axis_index("c")
    sid = lax.axis_index("s")
    wid = cid * _NS + sid

    @pl.when(sid == 0)
    def _():
        pltpu.sync_copy(z_hbm, shared)

    plsc.subcore_barrier()
    base_g = wid * _GPW

    def batch_body(b, _):
        gb = base_g + b * 16
        pltpu.sync_copy(r2d_hbm.at[pl.ds(gb, 16)], idxbuf)
        pltpu.sync_copy(p_hbm.at[pl.ds(gb * _GRP, 16 * _GRP)], pbuf)
        copies = []
        for j in range(16):
            copies.append(pltpu.async_copy(
                pbuf.at[pl.ds(j * _GRP, _GRP)], shared.at[idxbuf.at[j]],
                sdma, add=True))
        for c in copies:
            c.wait()
        return 0

    lax.fori_loop(0, _GPW // 16, batch_body, 0)
    plsc.subcore_barrier()

    @pl.when(sid == 0)
    def _():
        pltpu.sync_copy(shared, acc_hbm.at[cid])


def _sc_scatter(P, row2d, zeros_n16):
    mesh = plsc.VectorSubcoreMesh(core_axis_name="c", subcore_axis_name="s",
                                  num_cores=_NC, num_subcores=_NS)
    f = functools.partial(
        pl.kernel,
        out_type=jax.ShapeDtypeStruct((_NC, N, 16), jnp.float32),
        mesh=mesh,
        scratch_types=[
            pltpu.VMEM((16 * _GRP, 16), jnp.float32),
            pltpu.VMEM((16, _GRP), jnp.int32),
            pltpu.VMEM_SHARED((N, 16), jnp.float32),
            pltpu.SemaphoreType.DMA,
        ],
        compiler_params=pltpu.CompilerParams(use_tc_tiling_on_sc=False),
    )(_sc_scatter_body)
    return f(P, row2d, zeros_n16)


_EB = 4096


def _stage3_body(g_ref, ea_ref, s_ref, me_ref, b1_ref, wd_ref, w2p_ref, b2p_ref,
                 p_ref):
    g = g_ref[...]
    ea = ea_ref[...]
    st = s_ref[...].T
    dx = st[0:1, :]
    dy = st[1:2, :]
    dz = st[2:3, :]
    r2 = dx * dx + dy * dy + dz * dz
    rn = jnp.sqrt(r2)
    r_col = rn.T
    z = (g + jnp.dot(ea, me_ref[...], preferred_element_type=jnp.float32)
         + b1_ref[...] + r_col * wd_ref[...])
    h = z * _sigmoid(z)
    m = jnp.dot(h, w2p_ref[...], preferred_element_type=jnp.float32) + b2p_ref[...]
    mt = m.T
    inv = 1.0 / jnp.maximum(rn, 1e-12)
    ux = dx * inv
    uy = dy * inv
    uz = dz * inv
    m0 = mt[0:1, :]
    m1 = mt[1:2, :]
    m2 = mt[2:3, :]
    eid = pl.program_id(0) * _EB + lax.broadcasted_iota(jnp.int32, (1, _EB), 1)
    valid = (eid < E).astype(jnp.float32)
    zeros = jnp.zeros_like(m0)
    pt = jnp.concatenate([
        valid * m0,
        valid * (m1 * ux), valid * (m1 * uy), valid * (m1 * uz),
        valid * (m2 * (_S3 * ux * uz)),
        valid * (m2 * (_S3 * ux * uy)),
        valid * (m2 * (uy * uy - 0.5 * (ux * ux + uz * uz))),
        valid * (m2 * (_S3 * uy * uz)),
        valid * (m2 * (0.5 * _S3 * (uz * uz - ux * ux))),
        valid,
        zeros, zeros, zeros, zeros, zeros, zeros,
    ], axis=0)
    p_ref[...] = pt.T


def _stage3(G, ea, S, me, b1v, wdv, w2p, b2p):
    grid = _E_PAD // _EB
    return pl.pallas_call(
        _stage3_body,
        grid=(grid,),
        in_specs=[
            pl.BlockSpec((_EB, HID), lambda i: (i, 0)),
            pl.BlockSpec((_EB, EA), lambda i: (i, 0)),
            pl.BlockSpec((_EB, 16), lambda i: (i, 0)),
            pl.BlockSpec((EA, HID), lambda i: (0, 0)),
            pl.BlockSpec((1, HID), lambda i: (0, 0)),
            pl.BlockSpec((1, HID), lambda i: (0, 0)),
            pl.BlockSpec((HID, 8), lambda i: (0, 0)),
            pl.BlockSpec((1, 8), lambda i: (0, 0)),
        ],
        out_specs=pl.BlockSpec((_EB, 16), lambda i: (i, 0)),
        out_shape=jax.ShapeDtypeStruct((_E_PAD, 16), jnp.float32),
        interpret=_INTERP,
    )(G, ea, S, me, b1v, wdv, w2p, b2p)


def _stage5_body(acc0_ref, acc1_ref, nsc_ref, out_ref):
    t = acc0_ref[...] + acc1_ref[...]
    ns = t[:, 0:9] / jnp.maximum(t[:, 9:10], 1.0)
    out_ref[...] = ns + nsc_ref[:, 0:9]


def _stage5(acc0, acc1, nsc):
    return pl.pallas_call(
        _stage5_body,
        out_shape=jax.ShapeDtypeStruct((N, 9), jnp.float32),
        interpret=_INTERP,
    )(acc0, acc1, nsc)


def kernel(node_feat, node_pos, edge_index, edge_attr, data_batch,
           W1, b1, W2, b2, Wc1, bc1, Wc2, bc2):
    row = edge_index[0]
    col = edge_index[1]

    mf = W1[:, 1:1 + HID].T
    mcol = W1[:, 1 + HID:1 + 2 * HID].T
    me = W1[:, 1 + 2 * HID:].T
    wdv = W1[:, 0].reshape(1, HID)
    b1v = b1.reshape(1, HID)
    w2p = jnp.zeros((HID, 8), jnp.float32).at[:, 0:3].set(W2.T)
    b2p = jnp.zeros((1, 8), jnp.float32).at[0, 0:3].set(b2)
    wc1t = Wc1.T
    bc1v = bc1.reshape(1, HID)
    wc2p = jnp.zeros((HID, 8), jnp.float32).at[:, 0:3].set(Wc2.T)
    bc2p = jnp.zeros((1, 8), jnp.float32).at[0, 0:3].set(bc2)

    posp = jnp.concatenate(
        [node_pos, jnp.ones((N, 1), jnp.float32)], axis=1)
    db2 = data_batch.reshape(N, 1)

    TA, TB, nsc = _stage1(node_feat, posp, db2, mf, mcol, wc1t, bc1v, wc2p, bc2p)

    pad_rows = _IDX_ROWS - E // _GRP
    row2d = jnp.pad(row.reshape(E // _GRP, _GRP), ((0, pad_rows), (0, 0)))
    pad_rows_g = _IDX_ROWS_G - E // _GRPG
    row2dg = jnp.pad(row.reshape(E // _GRPG, _GRPG), ((0, pad_rows_g), (0, 0)))
    col2dg = jnp.pad(col.reshape(E // _GRPG, _GRPG), ((0, pad_rows_g), (0, 0)))
    eap = jnp.pad(edge_attr, ((0, _E_PAD - E), (0, 0)))
    posp16 = jnp.pad(node_pos, ((0, 0), (0, 13)))
    negp16 = jnp.pad(-node_pos, ((0, 0), (0, 13)))

    G, S = _sc_gather(TA, TB, posp16, negp16, row2dg, col2dg)

    P = _stage3(G, eap, S, me, b1v, wdv, w2p, b2p)

    acc = _sc_scatter(P, row2d, jnp.zeros((N, 16), jnp.float32))

    return _stage5(acc[0], acc[1], nsc)
